# baseline (device time: 154197 ns/iter reference)
import functools

import jax
import jax.numpy as jnp
from jax import lax
from jax.experimental import pallas as pl
from jax.experimental.pallas import tpu as pltpu

N_DEV = 4
SQ = 512
D = 1024
HQ = 8
DH = 128
SKV = 2048
SCALE = 0.08838834764831843
FINAL_SPLIT = 2


def kernel(x, Wq, Wo, K_ext, V_ext):
    x2 = x.reshape(SQ, D)

    def body(x_ref, wq_ref, wo_ref, k_hbm, v_hbm, out_ref,
             xg_ref, pown_ref, acc_ref, rs_send_ref, rs_recv_ref,
             o_buf_ref, k_buf, v_buf,
             ag_send_sems, ag_recv_sems, rs_send_sems, rs_recv_sems,
             kv_sems):
        my = lax.axis_index("i")
        left = lax.rem(my + N_DEV - 1, N_DEV)
        right = lax.rem(my + 1, N_DEV)

        barrier_sem = pltpu.get_barrier_semaphore()
        for nbr in (left, right):
            pl.semaphore_signal(barrier_sem, inc=1, device_id=(nbr,),
                                device_id_type=pl.DeviceIdType.MESH)
        pl.semaphore_wait(barrier_sem, 2)

        b_of = [lax.rem(my - h + N_DEV, N_DEV) for h in range(N_DEV)]

        dma_list = []
        for h in range(N_DEV):
            for _ in range(FINAL_SPLIT if h == N_DEV - 1 else 1):
                for hh in range(HQ):
                    dma_list.append((h, hh))
        pending = [None, None]
        jc = [0]

        def kv_issue(j):
            h, hh = dma_list[j]
            slot = j % 2
            hq = my * HQ + hh
            kc = pltpu.make_async_copy(
                k_hbm.at[b_of[h], :, hq, :], k_buf.at[slot],
                kv_sems.at[slot, 0])
            vc = pltpu.make_async_copy(
                v_hbm.at[b_of[h], :, hq, :], v_buf.at[slot],
                kv_sems.at[slot, 1])
            kc.start()
            vc.start()
            pending[slot] = (kc, vc)

        kv_issue(0)

        def attn_rows(x_val, dst_ref, r0, nr):
            q = jnp.dot(x_val[r0:r0 + nr, :], wq_ref[:, :],
                        preferred_element_type=jnp.float32)
            for hh in range(HQ):
                j = jc[0]
                jc[0] += 1
                slot = j % 2
                kc, vc = pending[slot]
                kc.wait()
                vc.wait()
                if j + 1 < len(dma_list):
                    kv_issue(j + 1)
                qh = q[:, hh * DH:(hh + 1) * DH]
                s = lax.dot_general(
                    qh, k_buf[slot, :, :], (((1,), (1,)), ((), ())),
                    preferred_element_type=jnp.float32) * SCALE
                p = jnp.exp(s)
                l = jnp.sum(p, axis=1, keepdims=True)
                o = jnp.dot(p, v_buf[slot, :, :],
                            preferred_element_type=jnp.float32) / l
                o_buf_ref[r0:r0 + nr, hh * DH:(hh + 1) * DH] = o
            dst_ref[r0:r0 + nr, :] = jnp.dot(
                o_buf_ref[r0:r0 + nr, :], wo_ref[:, :],
                preferred_element_type=jnp.float32)

        xg_ref[0, :, :] = x_ref[:, :]

        ag_sends = []
        snd = pltpu.make_async_remote_copy(
            src_ref=xg_ref.at[0], dst_ref=xg_ref.at[1],
            send_sem=ag_send_sems.at[0], recv_sem=ag_recv_sems.at[0],
            device_id=(right,), device_id_type=pl.DeviceIdType.MESH)
        snd.start()
        ag_sends.append(snd)

        attn_rows(x_ref[:, :], pown_ref, 0, SQ)

        nr = SQ // FINAL_SPLIT
        rs_rdmas = [None] * (N_DEV - 2)
        half_rdmas = []
        for h in range(1, N_DEV):
            rcv = pltpu.make_async_remote_copy(
                src_ref=xg_ref.at[h], dst_ref=xg_ref.at[h],
                send_sem=ag_send_sems.at[h - 1],
                recv_sem=ag_recv_sems.at[h - 1],
                device_id=(left,), device_id_type=pl.DeviceIdType.MESH)
            rcv.wait_recv()
            if h < N_DEV - 1:
                snd = pltpu.make_async_remote_copy(
                    src_ref=xg_ref.at[h], dst_ref=xg_ref.at[h + 1],
                    send_sem=ag_send_sems.at[h],
                    recv_sem=ag_recv_sems.at[h],
                    device_id=(right,), device_id_type=pl.DeviceIdType.MESH)
                snd.start()
                ag_sends.append(snd)

            s = h - 1
            if h < N_DEV - 1:
                if s == 0:
                    attn_rows(xg_ref[h, :, :], rs_send_ref, 0, SQ)
                else:
                    attn_rows(xg_ref[h, :, :], acc_ref, 0, SQ)
                    prev = pltpu.make_async_remote_copy(
                        src_ref=rs_send_ref, dst_ref=rs_recv_ref.at[s - 1],
                        send_sem=rs_send_sems.at[s - 1],
                        recv_sem=rs_recv_sems.at[s - 1],
                        device_id=(left,), device_id_type=pl.DeviceIdType.MESH)
                    prev.wait_recv()
                    rs_rdmas[s - 1].wait_send()
                    rs_send_ref[:, :] = (acc_ref[:, :]
                                         + rs_recv_ref[s - 1, :, :])
                rs = pltpu.make_async_remote_copy(
                    src_ref=rs_send_ref, dst_ref=rs_recv_ref.at[s],
                    send_sem=rs_send_sems.at[s], recv_sem=rs_recv_sems.at[s],
                    device_id=(right,), device_id_type=pl.DeviceIdType.MESH)
                rs.start()
                rs_rdmas[s] = rs
            else:
                for rb in range(FINAL_SPLIT):
                    r0 = rb * nr
                    attn_rows(xg_ref[h, :, :], acc_ref, r0, nr)
                    if rb == 0:
                        prev = pltpu.make_async_remote_copy(
                            src_ref=rs_send_ref,
                            dst_ref=rs_recv_ref.at[s - 1],
                            send_sem=rs_send_sems.at[s - 1],
                            recv_sem=rs_recv_sems.at[s - 1],
                            device_id=(left,),
                            device_id_type=pl.DeviceIdType.MESH)
                        prev.wait_recv()
                        rs_rdmas[s - 1].wait_send()
                    rs_send_ref[r0:r0 + nr, :] = (
                        acc_ref[r0:r0 + nr, :]
                        + rs_recv_ref[s - 1, r0:r0 + nr, :])
                    half = pltpu.make_async_remote_copy(
                        src_ref=rs_send_ref.at[pl.ds(r0, nr)],
                        dst_ref=rs_recv_ref.at[s, pl.ds(r0, nr)],
                        send_sem=rs_send_sems.at[N_DEV - 2 + rb],
                        recv_sem=rs_recv_sems.at[N_DEV - 2 + rb],
                        device_id=(right,),
                        device_id_type=pl.DeviceIdType.MESH)
                    half.start()
                    half_rdmas.append(half)

        for rb in range(FINAL_SPLIT):
            r0 = rb * nr
            fin = pltpu.make_async_remote_copy(
                src_ref=rs_send_ref.at[pl.ds(r0, nr)],
                dst_ref=rs_recv_ref.at[N_DEV - 2, pl.ds(r0, nr)],
                send_sem=rs_send_sems.at[N_DEV - 2 + rb],
                recv_sem=rs_recv_sems.at[N_DEV - 2 + rb],
                device_id=(left,), device_id_type=pl.DeviceIdType.MESH)
            fin.wait_recv()
            out_ref[r0:r0 + nr, :] = (rs_recv_ref[N_DEV - 2, r0:r0 + nr, :]
                                      + pown_ref[r0:r0 + nr, :])

        for half in half_rdmas:
            half.wait_send()
        for snd in ag_sends:
            snd.wait_send()

        @functools.partial(pl.run_scoped,
                           second_barrier=pltpu.SemaphoreType.REGULAR)
        def _(second_barrier):
            for nbr in (left, right):
                pl.semaphore_signal(second_barrier, inc=1, device_id=(nbr,),
                                    device_id_type=pl.DeviceIdType.MESH)
            pl.semaphore_wait(second_barrier, 2)

    out = pl.pallas_call(
        body,
        out_shape=jax.ShapeDtypeStruct((SQ, D), jnp.float32),
        in_specs=[
            pl.BlockSpec(memory_space=pltpu.VMEM),
            pl.BlockSpec(memory_space=pltpu.VMEM),
            pl.BlockSpec(memory_space=pltpu.VMEM),
            pl.BlockSpec(memory_space=pl.ANY),
            pl.BlockSpec(memory_space=pl.ANY),
        ],
        out_specs=pl.BlockSpec(memory_space=pltpu.VMEM),
        scratch_shapes=[
            pltpu.VMEM((N_DEV, SQ, D), jnp.float32),
            pltpu.VMEM((SQ, D), jnp.float32),
            pltpu.VMEM((SQ, D), jnp.float32),
            pltpu.VMEM((SQ, D), jnp.float32),
            pltpu.VMEM((N_DEV - 1, SQ, D), jnp.float32),
            pltpu.VMEM((SQ, D), jnp.float32),
            pltpu.VMEM((2, SKV, DH), jnp.float32),
            pltpu.VMEM((2, SKV, DH), jnp.float32),
            pltpu.SemaphoreType.DMA((N_DEV - 1,)),
            pltpu.SemaphoreType.DMA((N_DEV - 1,)),
            pltpu.SemaphoreType.DMA((N_DEV - 2 + FINAL_SPLIT,)),
            pltpu.SemaphoreType.DMA((N_DEV - 2 + FINAL_SPLIT,)),
            pltpu.SemaphoreType.DMA((2, 2)),
        ],
        compiler_params=pltpu.CompilerParams(collective_id=0),
    )(x2, Wq, Wo, K_ext, V_ext)
    return out.reshape(1, SQ, D)


# device time: 124707 ns/iter; 1.2365x vs baseline; 1.2365x over previous
import functools

import jax
import jax.numpy as jnp
from jax import lax
from jax.experimental import pallas as pl
from jax.experimental.pallas import tpu as pltpu

N_DEV = 4
SQ = 512
D = 1024
HQ = 8
DH = 128
SKV = 2048
SCALE = 0.08838834764831843
HR = SQ // 2


def kernel(x, Wq, Wo, K_ext, V_ext):
    x2 = x.reshape(SQ, D)

    def body(x_ref, wq_ref, wo_ref, k_hbm, v_hbm, out_ref,
             xga_ref, xgb_ref, pown_ref, acc_ref, o_buf_ref,
             rsa_send_ref, rsb_send_ref, rsa_recv_ref, rsb_recv_ref,
             k_buf, v_buf,
             aga_send_sems, aga_recv_sems, agb_send_sems, agb_recv_sems,
             rsa_send_sems, rsa_recv_sems, rsb_send_sems, rsb_recv_sems,
             kv_sems):
        my = lax.axis_index("i")
        left = lax.rem(my + N_DEV - 1, N_DEV)
        right = lax.rem(my + 1, N_DEV)

        barrier_sem = pltpu.get_barrier_semaphore()
        for nbr in (left, right):
            pl.semaphore_signal(barrier_sem, inc=1, device_id=(nbr,),
                                device_id_type=pl.DeviceIdType.MESH)
        pl.semaphore_wait(barrier_sem, 2)

        b_a = [lax.rem(my - h + N_DEV, N_DEV) for h in range(N_DEV)]
        b_b = [lax.rem(my + h, N_DEV) for h in range(N_DEV)]

        dma_list = [(b_a[0], hh) for hh in range(HQ)]
        for h in range(1, N_DEV):
            dma_list += [(b_a[h], hh) for hh in range(HQ)]
            dma_list += [(b_b[h], hh) for hh in range(HQ)]
        pending = [None, None]
        jc = [0]

        def kv_issue(j):
            b, hh = dma_list[j]
            slot = j % 2
            hq = my * HQ + hh
            kc = pltpu.make_async_copy(
                k_hbm.at[b, :, hq, :], k_buf.at[slot],
                kv_sems.at[slot, 0])
            vc = pltpu.make_async_copy(
                v_hbm.at[b, :, hq, :], v_buf.at[slot],
                kv_sems.at[slot, 1])
            kc.start()
            vc.start()
            pending[slot] = (kc, vc)

        kv_issue(0)

        def attn_block(x_rows, dst_ref, r0, nr):
            q = jnp.dot(x_rows, wq_ref[:, :],
                        preferred_element_type=jnp.float32)
            for hh in range(HQ):
                j = jc[0]
                jc[0] += 1
                slot = j % 2
                kc, vc = pending[slot]
                kc.wait()
                vc.wait()
                if j + 1 < len(dma_list):
                    kv_issue(j + 1)
                qh = q[:, hh * DH:(hh + 1) * DH]
                s = lax.dot_general(
                    qh, k_buf[slot, :, :], (((1,), (1,)), ((), ())),
                    preferred_element_type=jnp.float32) * SCALE
                p = jnp.exp(s)
                l = jnp.sum(p, axis=1, keepdims=True)
                o = jnp.dot(p, v_buf[slot, :, :],
                            preferred_element_type=jnp.float32) / l
                o_buf_ref[r0:r0 + nr, hh * DH:(hh + 1) * DH] = o
            dst_ref[r0:r0 + nr, :] = jnp.dot(
                o_buf_ref[r0:r0 + nr, :], wo_ref[:, :],
                preferred_element_type=jnp.float32)

        xga_ref[0, :, :] = x_ref[0:HR, :]
        xgb_ref[0, :, :] = x_ref[HR:SQ, :]
        sends = []
        snd = pltpu.make_async_remote_copy(
            src_ref=xga_ref.at[0], dst_ref=xga_ref.at[1],
            send_sem=aga_send_sems.at[0], recv_sem=aga_recv_sems.at[0],
            device_id=(right,), device_id_type=pl.DeviceIdType.MESH)
        snd.start()
        sends.append(snd)
        snd = pltpu.make_async_remote_copy(
            src_ref=xgb_ref.at[0], dst_ref=xgb_ref.at[1],
            send_sem=agb_send_sems.at[0], recv_sem=agb_recv_sems.at[0],
            device_id=(left,), device_id_type=pl.DeviceIdType.MESH)
        snd.start()
        sends.append(snd)

        attn_block(x_ref[:, :], pown_ref, 0, SQ)

        rsa_rdmas = [None] * (N_DEV - 1)
        rsb_rdmas = [None] * (N_DEV - 1)
        for h in range(1, N_DEV):
            s = h - 1
            rcv = pltpu.make_async_remote_copy(
                src_ref=xga_ref.at[h], dst_ref=xga_ref.at[h],
                send_sem=aga_send_sems.at[h - 1],
                recv_sem=aga_recv_sems.at[h - 1],
                device_id=(left,), device_id_type=pl.DeviceIdType.MESH)
            rcv.wait_recv()
            if h < N_DEV - 1:
                snd = pltpu.make_async_remote_copy(
                    src_ref=xga_ref.at[h], dst_ref=xga_ref.at[h + 1],
                    send_sem=aga_send_sems.at[h],
                    recv_sem=aga_recv_sems.at[h],
                    device_id=(right,), device_id_type=pl.DeviceIdType.MESH)
                snd.start()
                sends.append(snd)
            attn_block(xga_ref[h, :, :], acc_ref, 0, HR)
            if s == 0:
                rsa_send_ref[:, :] = acc_ref[0:HR, :]
            else:
                prev = pltpu.make_async_remote_copy(
                    src_ref=rsa_send_ref, dst_ref=rsa_recv_ref.at[s - 1],
                    send_sem=rsa_send_sems.at[s - 1],
                    recv_sem=rsa_recv_sems.at[s - 1],
                    device_id=(left,), device_id_type=pl.DeviceIdType.MESH)
                prev.wait_recv()
                rsa_rdmas[s - 1].wait_send()
                rsa_send_ref[:, :] = (acc_ref[0:HR, :]
                                      + rsa_recv_ref[s - 1, :, :])
            rs = pltpu.make_async_remote_copy(
                src_ref=rsa_send_ref, dst_ref=rsa_recv_ref.at[s],
                send_sem=rsa_send_sems.at[s], recv_sem=rsa_recv_sems.at[s],
                device_id=(right,), device_id_type=pl.DeviceIdType.MESH)
            rs.start()
            rsa_rdmas[s] = rs

            rcv = pltpu.make_async_remote_copy(
                src_ref=xgb_ref.at[h], dst_ref=xgb_ref.at[h],
                send_sem=agb_send_sems.at[h - 1],
                recv_sem=agb_recv_sems.at[h - 1],
                device_id=(right,), device_id_type=pl.DeviceIdType.MESH)
            rcv.wait_recv()
            if h < N_DEV - 1:
                snd = pltpu.make_async_remote_copy(
                    src_ref=xgb_ref.at[h], dst_ref=xgb_ref.at[h + 1],
                    send_sem=agb_send_sems.at[h],
                    recv_sem=agb_recv_sems.at[h],
                    device_id=(left,), device_id_type=pl.DeviceIdType.MESH)
                snd.start()
                sends.append(snd)
            attn_block(xgb_ref[h, :, :], acc_ref, HR, HR)
            if s == 0:
                rsb_send_ref[:, :] = acc_ref[HR:SQ, :]
            else:
                prev = pltpu.make_async_remote_copy(
                    src_ref=rsb_send_ref, dst_ref=rsb_recv_ref.at[s - 1],
                    send_sem=rsb_send_sems.at[s - 1],
                    recv_sem=rsb_recv_sems.at[s - 1],
                    device_id=(right,), device_id_type=pl.DeviceIdType.MESH)
                prev.wait_recv()
                rsb_rdmas[s - 1].wait_send()
                rsb_send_ref[:, :] = (acc_ref[HR:SQ, :]
                                      + rsb_recv_ref[s - 1, :, :])
            rs = pltpu.make_async_remote_copy(
                src_ref=rsb_send_ref, dst_ref=rsb_recv_ref.at[s],
                send_sem=rsb_send_sems.at[s], recv_sem=rsb_recv_sems.at[s],
                device_id=(left,), device_id_type=pl.DeviceIdType.MESH)
            rs.start()
            rsb_rdmas[s] = rs

        fin = pltpu.make_async_remote_copy(
            src_ref=rsa_send_ref, dst_ref=rsa_recv_ref.at[N_DEV - 2],
            send_sem=rsa_send_sems.at[N_DEV - 2],
            recv_sem=rsa_recv_sems.at[N_DEV - 2],
            device_id=(left,), device_id_type=pl.DeviceIdType.MESH)
        fin.wait_recv()
        out_ref[0:HR, :] = (rsa_recv_ref[N_DEV - 2, :, :]
                            + pown_ref[0:HR, :])
        fin = pltpu.make_async_remote_copy(
            src_ref=rsb_send_ref, dst_ref=rsb_recv_ref.at[N_DEV - 2],
            send_sem=rsb_send_sems.at[N_DEV - 2],
            recv_sem=rsb_recv_sems.at[N_DEV - 2],
            device_id=(right,), device_id_type=pl.DeviceIdType.MESH)
        fin.wait_recv()
        out_ref[HR:SQ, :] = (rsb_recv_ref[N_DEV - 2, :, :]
                             + pown_ref[HR:SQ, :])

        rsa_rdmas[N_DEV - 2].wait_send()
        rsb_rdmas[N_DEV - 2].wait_send()
        for snd in sends:
            snd.wait_send()

        @functools.partial(pl.run_scoped,
                           second_barrier=pltpu.SemaphoreType.REGULAR)
        def _(second_barrier):
            for nbr in (left, right):
                pl.semaphore_signal(second_barrier, inc=1, device_id=(nbr,),
                                    device_id_type=pl.DeviceIdType.MESH)
            pl.semaphore_wait(second_barrier, 2)

    out = pl.pallas_call(
        body,
        out_shape=jax.ShapeDtypeStruct((SQ, D), jnp.float32),
        in_specs=[
            pl.BlockSpec(memory_space=pltpu.VMEM),
            pl.BlockSpec(memory_space=pltpu.VMEM),
            pl.BlockSpec(memory_space=pltpu.VMEM),
            pl.BlockSpec(memory_space=pl.ANY),
            pl.BlockSpec(memory_space=pl.ANY),
        ],
        out_specs=pl.BlockSpec(memory_space=pltpu.VMEM),
        scratch_shapes=[
            pltpu.VMEM((N_DEV, HR, D), jnp.float32),
            pltpu.VMEM((N_DEV, HR, D), jnp.float32),
            pltpu.VMEM((SQ, D), jnp.float32),
            pltpu.VMEM((SQ, D), jnp.float32),
            pltpu.VMEM((SQ, D), jnp.float32),
            pltpu.VMEM((HR, D), jnp.float32),
            pltpu.VMEM((HR, D), jnp.float32),
            pltpu.VMEM((N_DEV - 1, HR, D), jnp.float32),
            pltpu.VMEM((N_DEV - 1, HR, D), jnp.float32),
            pltpu.VMEM((2, SKV, DH), jnp.float32),
            pltpu.VMEM((2, SKV, DH), jnp.float32),
            pltpu.SemaphoreType.DMA((N_DEV - 1,)),
            pltpu.SemaphoreType.DMA((N_DEV - 1,)),
            pltpu.SemaphoreType.DMA((N_DEV - 1,)),
            pltpu.SemaphoreType.DMA((N_DEV - 1,)),
            pltpu.SemaphoreType.DMA((N_DEV - 1,)),
            pltpu.SemaphoreType.DMA((N_DEV - 1,)),
            pltpu.SemaphoreType.DMA((N_DEV - 1,)),
            pltpu.SemaphoreType.DMA((N_DEV - 1,)),
            pltpu.SemaphoreType.DMA((2, 2)),
        ],
        compiler_params=pltpu.CompilerParams(collective_id=0),
    )(x2, Wq, Wo, K_ext, V_ext)
    return out.reshape(1, SQ, D)


# device time: 96403 ns/iter; 1.5995x vs baseline; 1.2936x over previous
import functools

import jax
import jax.numpy as jnp
from jax import lax
from jax.experimental import pallas as pl
from jax.experimental.pallas import tpu as pltpu

N_DEV = 4
SQ = 512
D = 1024
HQ = 8
DH = 128
SKV = 2048
SCALE = 0.08838834764831843
HR = SQ // 2


def kernel(x, Wq, Wo, K_ext, V_ext):
    x2 = x.reshape(SQ, D)

    def body(x_ref, wq_ref, wo_ref, k_hbm, v_hbm, out_ref,
             xga_ref, xgb_ref, pown_ref, acc_ref, o_buf_ref,
             rsa_send_ref, rsb_send_ref, rsa_recv_ref, rsb_recv_ref,
             k_buf, v_buf,
             aga_send_sems, aga_recv_sems, agb_send_sems, agb_recv_sems,
             rsa_send_sems, rsa_recv_sems, rsb_send_sems, rsb_recv_sems,
             kv_sems):
        my = lax.axis_index("i")
        left = lax.rem(my + N_DEV - 1, N_DEV)
        right = lax.rem(my + 1, N_DEV)

        barrier_sem = pltpu.get_barrier_semaphore()
        for nbr in (left, right):
            pl.semaphore_signal(barrier_sem, inc=1, device_id=(nbr,),
                                device_id_type=pl.DeviceIdType.MESH)
        pl.semaphore_wait(barrier_sem, 2)

        b_a = [lax.rem(my - h + N_DEV, N_DEV) for h in range(N_DEV)]
        b_b = [lax.rem(my + h, N_DEV) for h in range(N_DEV)]

        dma_list = [(b_a[0], hh) for hh in range(HQ)]
        for h in range(1, N_DEV):
            dma_list += [(b_a[h], hh) for hh in range(HQ)]
            reps = 2 if h == N_DEV - 1 else 1
            for _ in range(reps):
                dma_list += [(b_b[h], hh) for hh in range(HQ)]
        NSLOT = 4
        pending = [None] * NSLOT
        jc = [0]

        def kv_issue(j):
            b, hh = dma_list[j]
            slot = j % NSLOT
            hq = my * HQ + hh
            kc = pltpu.make_async_copy(
                k_hbm.at[b, :, hq, :], k_buf.at[slot],
                kv_sems.at[slot, 0])
            vc = pltpu.make_async_copy(
                v_hbm.at[b, :, hq, :], v_buf.at[slot],
                kv_sems.at[slot, 1])
            kc.start()
            vc.start()
            pending[slot] = (kc, vc)

        for _j in range(3):
            kv_issue(_j)

        def attn_block(x_rows, dst_ref, r0, nr):
            q = jnp.dot(x_rows, wq_ref[:, :],
                        preferred_element_type=jnp.float32)
            for hh in range(HQ):
                j = jc[0]
                jc[0] += 1
                slot = j % NSLOT
                kc, vc = pending[slot]
                kc.wait()
                vc.wait()
                if j + 3 < len(dma_list):
                    kv_issue(j + 3)
                qh = q[:, hh * DH:(hh + 1) * DH]
                s = lax.dot_general(
                    qh, k_buf[slot, :, :], (((1,), (1,)), ((), ())),
                    preferred_element_type=jnp.float32) * SCALE
                p = jnp.exp(s)
                l = jnp.sum(p, axis=1, keepdims=True)
                o = jnp.dot(p, v_buf[slot, :, :],
                            preferred_element_type=jnp.float32) / l
                o_buf_ref[r0:r0 + nr, hh * DH:(hh + 1) * DH] = o
            dst_ref[r0:r0 + nr, :] = jnp.dot(
                o_buf_ref[r0:r0 + nr, :], wo_ref[:, :],
                preferred_element_type=jnp.float32)

        sends = []
        snd = pltpu.make_async_remote_copy(
            src_ref=x_ref.at[pl.ds(0, HR)], dst_ref=xga_ref.at[0],
            send_sem=aga_send_sems.at[0], recv_sem=aga_recv_sems.at[0],
            device_id=(right,), device_id_type=pl.DeviceIdType.MESH)
        snd.start()
        sends.append(snd)
        snd = pltpu.make_async_remote_copy(
            src_ref=x_ref.at[pl.ds(HR, HR)], dst_ref=xgb_ref.at[0],
            send_sem=agb_send_sems.at[0], recv_sem=agb_recv_sems.at[0],
            device_id=(left,), device_id_type=pl.DeviceIdType.MESH)
        snd.start()
        sends.append(snd)

        attn_block(x_ref[:, :], pown_ref, 0, SQ)

        rsa_rdmas = [None] * (N_DEV - 1)
        rsb_rdmas = [None] * (N_DEV - 1)
        for h in range(1, N_DEV):
            s = h - 1
            rcv = pltpu.make_async_remote_copy(
                src_ref=xga_ref.at[h - 1], dst_ref=xga_ref.at[h - 1],
                send_sem=aga_send_sems.at[h - 1],
                recv_sem=aga_recv_sems.at[h - 1],
                device_id=(left,), device_id_type=pl.DeviceIdType.MESH)
            rcv.wait_recv()
            if h < N_DEV - 1:
                snd = pltpu.make_async_remote_copy(
                    src_ref=xga_ref.at[h - 1], dst_ref=xga_ref.at[h],
                    send_sem=aga_send_sems.at[h],
                    recv_sem=aga_recv_sems.at[h],
                    device_id=(right,), device_id_type=pl.DeviceIdType.MESH)
                snd.start()
                sends.append(snd)
            attn_block(xga_ref[h - 1, :, :], acc_ref, 0, HR)
            if s == 0:
                rsa_send_ref[:, :] = acc_ref[0:HR, :]
            else:
                prev = pltpu.make_async_remote_copy(
                    src_ref=rsa_send_ref, dst_ref=rsa_recv_ref.at[s - 1],
                    send_sem=rsa_send_sems.at[s - 1],
                    recv_sem=rsa_recv_sems.at[s - 1],
                    device_id=(left,), device_id_type=pl.DeviceIdType.MESH)
                prev.wait_recv()
                rsa_rdmas[s - 1].wait_send()
                rsa_send_ref[:, :] = (acc_ref[0:HR, :]
                                      + rsa_recv_ref[s - 1, :, :])
            rs = pltpu.make_async_remote_copy(
                src_ref=rsa_send_ref, dst_ref=rsa_recv_ref.at[s],
                send_sem=rsa_send_sems.at[s], recv_sem=rsa_recv_sems.at[s],
                device_id=(right,), device_id_type=pl.DeviceIdType.MESH)
            rs.start()
            rsa_rdmas[s] = rs

            rcv = pltpu.make_async_remote_copy(
                src_ref=xgb_ref.at[h - 1], dst_ref=xgb_ref.at[h - 1],
                send_sem=agb_send_sems.at[h - 1],
                recv_sem=agb_recv_sems.at[h - 1],
                device_id=(right,), device_id_type=pl.DeviceIdType.MESH)
            rcv.wait_recv()
            if h < N_DEV - 1:
                snd = pltpu.make_async_remote_copy(
                    src_ref=xgb_ref.at[h - 1], dst_ref=xgb_ref.at[h],
                    send_sem=agb_send_sems.at[h],
                    recv_sem=agb_recv_sems.at[h],
                    device_id=(left,), device_id_type=pl.DeviceIdType.MESH)
                snd.start()
                sends.append(snd)
                attn_block(xgb_ref[h - 1, :, :], acc_ref, HR, HR)
                if s == 0:
                    rsb_send_ref[:, :] = acc_ref[HR:SQ, :]
                else:
                    prev = pltpu.make_async_remote_copy(
                        src_ref=rsb_send_ref,
                        dst_ref=rsb_recv_ref.at[s - 1],
                        send_sem=rsb_send_sems.at[s - 1],
                        recv_sem=rsb_recv_sems.at[s - 1],
                        device_id=(right,),
                        device_id_type=pl.DeviceIdType.MESH)
                    prev.wait_recv()
                    rsb_rdmas[s - 1].wait_send()
                    rsb_send_ref[:, :] = (acc_ref[HR:SQ, :]
                                          + rsb_recv_ref[s - 1, :, :])
                rs = pltpu.make_async_remote_copy(
                    src_ref=rsb_send_ref, dst_ref=rsb_recv_ref.at[s],
                    send_sem=rsb_send_sems.at[s],
                    recv_sem=rsb_recv_sems.at[s],
                    device_id=(left,), device_id_type=pl.DeviceIdType.MESH)
                rs.start()
                rsb_rdmas[s] = rs
            else:
                HB = HR // 2
                for rb in range(2):
                    lo = rb * HB
                    attn_block(xgb_ref[h - 1, lo:lo + HB, :], acc_ref,
                               HR + lo, HB)
                    if rb == 0:
                        prev = pltpu.make_async_remote_copy(
                            src_ref=rsb_send_ref,
                            dst_ref=rsb_recv_ref.at[s - 1],
                            send_sem=rsb_send_sems.at[s - 1],
                            recv_sem=rsb_recv_sems.at[s - 1],
                            device_id=(right,),
                            device_id_type=pl.DeviceIdType.MESH)
                        prev.wait_recv()
                        rsb_rdmas[s - 1].wait_send()
                    rsb_send_ref[lo:lo + HB, :] = (
                        acc_ref[HR + lo:HR + lo + HB, :]
                        + rsb_recv_ref[s - 1, lo:lo + HB, :])
                    half = pltpu.make_async_remote_copy(
                        src_ref=rsb_send_ref.at[pl.ds(lo, HB)],
                        dst_ref=rsb_recv_ref.at[s, pl.ds(lo, HB)],
                        send_sem=rsb_send_sems.at[s + rb],
                        recv_sem=rsb_recv_sems.at[s + rb],
                        device_id=(left,),
                        device_id_type=pl.DeviceIdType.MESH)
                    half.start()
                    rsb_rdmas.append(half)

        fin = pltpu.make_async_remote_copy(
            src_ref=rsa_send_ref, dst_ref=rsa_recv_ref.at[N_DEV - 2],
            send_sem=rsa_send_sems.at[N_DEV - 2],
            recv_sem=rsa_recv_sems.at[N_DEV - 2],
            device_id=(left,), device_id_type=pl.DeviceIdType.MESH)
        fin.wait_recv()
        out_ref[0:HR, :] = (rsa_recv_ref[N_DEV - 2, :, :]
                            + pown_ref[0:HR, :])
        HB = HR // 2
        for rb in range(2):
            lo = rb * HB
            fin = pltpu.make_async_remote_copy(
                src_ref=rsb_send_ref.at[pl.ds(lo, HB)],
                dst_ref=rsb_recv_ref.at[N_DEV - 2, pl.ds(lo, HB)],
                send_sem=rsb_send_sems.at[N_DEV - 2 + rb],
                recv_sem=rsb_recv_sems.at[N_DEV - 2 + rb],
                device_id=(right,), device_id_type=pl.DeviceIdType.MESH)
            fin.wait_recv()
            out_ref[HR + lo:HR + lo + HB, :] = (
                rsb_recv_ref[N_DEV - 2, lo:lo + HB, :]
                + pown_ref[HR + lo:HR + lo + HB, :])

        rsa_rdmas[N_DEV - 2].wait_send()
        rsb_rdmas[-2].wait_send()
        rsb_rdmas[-1].wait_send()
        for snd in sends:
            snd.wait_send()

        @functools.partial(pl.run_scoped,
                           second_barrier=pltpu.SemaphoreType.REGULAR)
        def _(second_barrier):
            for nbr in (left, right):
                pl.semaphore_signal(second_barrier, inc=1, device_id=(nbr,),
                                    device_id_type=pl.DeviceIdType.MESH)
            pl.semaphore_wait(second_barrier, 2)

    out = pl.pallas_call(
        body,
        out_shape=jax.ShapeDtypeStruct((SQ, D), jnp.float32),
        in_specs=[
            pl.BlockSpec(memory_space=pltpu.VMEM),
            pl.BlockSpec(memory_space=pltpu.VMEM),
            pl.BlockSpec(memory_space=pltpu.VMEM),
            pl.BlockSpec(memory_space=pl.ANY),
            pl.BlockSpec(memory_space=pl.ANY),
        ],
        out_specs=pl.BlockSpec(memory_space=pltpu.VMEM),
        scratch_shapes=[
            pltpu.VMEM((N_DEV - 1, HR, D), jnp.float32),
            pltpu.VMEM((N_DEV - 1, HR, D), jnp.float32),
            pltpu.VMEM((SQ, D), jnp.float32),
            pltpu.VMEM((SQ, D), jnp.float32),
            pltpu.VMEM((SQ, D), jnp.float32),
            pltpu.VMEM((HR, D), jnp.float32),
            pltpu.VMEM((HR, D), jnp.float32),
            pltpu.VMEM((N_DEV - 1, HR, D), jnp.float32),
            pltpu.VMEM((N_DEV - 1, HR, D), jnp.float32),
            pltpu.VMEM((4, SKV, DH), jnp.float32),
            pltpu.VMEM((4, SKV, DH), jnp.float32),
            pltpu.SemaphoreType.DMA((N_DEV - 1,)),
            pltpu.SemaphoreType.DMA((N_DEV - 1,)),
            pltpu.SemaphoreType.DMA((N_DEV - 1,)),
            pltpu.SemaphoreType.DMA((N_DEV - 1,)),
            pltpu.SemaphoreType.DMA((N_DEV - 1,)),
            pltpu.SemaphoreType.DMA((N_DEV - 1,)),
            pltpu.SemaphoreType.DMA((N_DEV,)),
            pltpu.SemaphoreType.DMA((N_DEV,)),
            pltpu.SemaphoreType.DMA((4, 2)),
        ],
        compiler_params=pltpu.CompilerParams(collective_id=0),
    )(x2, Wq, Wo, K_ext, V_ext)
    return out.reshape(1, SQ, D)


# device time: 93155 ns/iter; 1.6553x vs baseline; 1.0349x over previous
import functools

import jax
import jax.numpy as jnp
from jax import lax
from jax.experimental import pallas as pl
from jax.experimental.pallas import tpu as pltpu

N_DEV = 4
SQ = 512
D = 1024
HQ = 8
DH = 128
SKV = 2048
SCALE = 0.08838834764831843
HR = SQ // 2


def kernel(x, Wq, Wo, K_ext, V_ext):
    x2 = x.reshape(SQ, D)

    def body(x_ref, wq_ref, wo_ref, k_hbm, v_hbm, out_ref,
             xga_ref, xgb_ref, pown_ref, acc_ref, o_buf_ref,
             rsa_send_ref, rsb_send_ref, rsa_recv_ref, rsb_recv_ref,
             fbf_send_ref, fbf_recv_ref, k_buf, v_buf,
             aga_send_sems, aga_recv_sems, agb_send_sems, agb_recv_sems,
             rsa_send_sems, rsa_recv_sems, rsb_send_sems, rsb_recv_sems,
             kv_sems):
        my = lax.axis_index("i")
        left = lax.rem(my + N_DEV - 1, N_DEV)
        right = lax.rem(my + 1, N_DEV)

        barrier_sem = pltpu.get_barrier_semaphore()
        for nbr in (left, right):
            pl.semaphore_signal(barrier_sem, inc=1, device_id=(nbr,),
                                device_id_type=pl.DeviceIdType.MESH)
        pl.semaphore_wait(barrier_sem, 2)

        b_a = [lax.rem(my - h + N_DEV, N_DEV) for h in range(N_DEV)]
        b_b = [lax.rem(my + h, N_DEV) for h in range(N_DEV)]

        dma_list = [(b_a[0], hh) for hh in range(HQ)]
        for h in range(1, N_DEV):
            dma_list += [(b_a[h], hh) for hh in range(HQ)]
            reps = 2 if h == N_DEV - 1 else 1
            for _ in range(reps):
                dma_list += [(b_b[h], hh) for hh in range(HQ)]
        NSLOT = 4
        pending = [None] * NSLOT
        jc = [0]

        def kv_issue(j):
            b, hh = dma_list[j]
            slot = j % NSLOT
            hq = my * HQ + hh
            kc = pltpu.make_async_copy(
                k_hbm.at[b, :, hq, :], k_buf.at[slot],
                kv_sems.at[slot, 0])
            vc = pltpu.make_async_copy(
                v_hbm.at[b, :, hq, :], v_buf.at[slot],
                kv_sems.at[slot, 1])
            kc.start()
            vc.start()
            pending[slot] = (kc, vc)

        for _j in range(3):
            kv_issue(_j)

        def attn_block(x_rows, dst_ref, r0, nr):
            q = jnp.dot(x_rows, wq_ref[:, :],
                        preferred_element_type=jnp.float32) * SCALE
            for hh in range(HQ):
                j = jc[0]
                jc[0] += 1
                slot = j % NSLOT
                kc, vc = pending[slot]
                kc.wait()
                vc.wait()
                if j + 3 < len(dma_list):
                    kv_issue(j + 3)
                qh = q[:, hh * DH:(hh + 1) * DH]
                s = lax.dot_general(
                    qh, k_buf[slot, :, :], (((1,), (1,)), ((), ())),
                    preferred_element_type=jnp.float32)
                p = jnp.exp(s)
                l = jnp.sum(p, axis=1, keepdims=True)
                o = jnp.dot(p, v_buf[slot, :, :],
                            preferred_element_type=jnp.float32) * (1.0 / l)
                o_buf_ref[r0:r0 + nr, hh * DH:(hh + 1) * DH] = o
            dst_ref[r0:r0 + nr, :] = jnp.dot(
                o_buf_ref[r0:r0 + nr, :], wo_ref[:, :],
                preferred_element_type=jnp.float32)

        sends = []
        snd = pltpu.make_async_remote_copy(
            src_ref=x_ref.at[pl.ds(0, HR)], dst_ref=xga_ref.at[0],
            send_sem=aga_send_sems.at[0], recv_sem=aga_recv_sems.at[0],
            device_id=(right,), device_id_type=pl.DeviceIdType.MESH)
        snd.start()
        sends.append(snd)
        snd = pltpu.make_async_remote_copy(
            src_ref=x_ref.at[pl.ds(HR, HR)], dst_ref=xgb_ref.at[0],
            send_sem=agb_send_sems.at[0], recv_sem=agb_recv_sems.at[0],
            device_id=(left,), device_id_type=pl.DeviceIdType.MESH)
        snd.start()
        sends.append(snd)

        attn_block(x_ref[:, :], pown_ref, 0, SQ)

        rsa_rdmas = [None] * (N_DEV - 1)
        rsb_rdmas = [None] * (N_DEV - 1)
        for h in range(1, N_DEV):
            s = h - 1
            rcv = pltpu.make_async_remote_copy(
                src_ref=xga_ref.at[h - 1], dst_ref=xga_ref.at[h - 1],
                send_sem=aga_send_sems.at[h - 1],
                recv_sem=aga_recv_sems.at[h - 1],
                device_id=(left,), device_id_type=pl.DeviceIdType.MESH)
            rcv.wait_recv()
            if h < N_DEV - 1:
                snd = pltpu.make_async_remote_copy(
                    src_ref=xga_ref.at[h - 1], dst_ref=xga_ref.at[h],
                    send_sem=aga_send_sems.at[h],
                    recv_sem=aga_recv_sems.at[h],
                    device_id=(right,), device_id_type=pl.DeviceIdType.MESH)
                snd.start()
                sends.append(snd)
            attn_block(xga_ref[h - 1, :, :], acc_ref, 0, HR)
            if s == 0:
                rsa_send_ref[:, :] = acc_ref[0:HR, :]
            else:
                prev = pltpu.make_async_remote_copy(
                    src_ref=rsa_send_ref, dst_ref=rsa_recv_ref.at[s - 1],
                    send_sem=rsa_send_sems.at[s - 1],
                    recv_sem=rsa_recv_sems.at[s - 1],
                    device_id=(left,), device_id_type=pl.DeviceIdType.MESH)
                prev.wait_recv()
                rsa_rdmas[s - 1].wait_send()
                rsa_send_ref[:, :] = (acc_ref[0:HR, :]
                                      + rsa_recv_ref[s - 1, :, :])
            rs = pltpu.make_async_remote_copy(
                src_ref=rsa_send_ref, dst_ref=rsa_recv_ref.at[s],
                send_sem=rsa_send_sems.at[s], recv_sem=rsa_recv_sems.at[s],
                device_id=(right,), device_id_type=pl.DeviceIdType.MESH)
            rs.start()
            rsa_rdmas[s] = rs

            rcv = pltpu.make_async_remote_copy(
                src_ref=xgb_ref.at[h - 1], dst_ref=xgb_ref.at[h - 1],
                send_sem=agb_send_sems.at[h - 1],
                recv_sem=agb_recv_sems.at[h - 1],
                device_id=(right,), device_id_type=pl.DeviceIdType.MESH)
            rcv.wait_recv()
            if h < N_DEV - 1:
                snd = pltpu.make_async_remote_copy(
                    src_ref=xgb_ref.at[h - 1], dst_ref=xgb_ref.at[h],
                    send_sem=agb_send_sems.at[h],
                    recv_sem=agb_recv_sems.at[h],
                    device_id=(left,), device_id_type=pl.DeviceIdType.MESH)
                snd.start()
                sends.append(snd)
                attn_block(xgb_ref[h - 1, :, :], acc_ref, HR, HR)
                if s == 0:
                    rsb_send_ref[:, :] = acc_ref[HR:SQ, :]
                else:
                    prev = pltpu.make_async_remote_copy(
                        src_ref=rsb_send_ref,
                        dst_ref=rsb_recv_ref.at[s - 1],
                        send_sem=rsb_send_sems.at[s - 1],
                        recv_sem=rsb_recv_sems.at[s - 1],
                        device_id=(right,),
                        device_id_type=pl.DeviceIdType.MESH)
                    prev.wait_recv()
                    rsb_rdmas[s - 1].wait_send()
                    rsb_send_ref[:, :] = (acc_ref[HR:SQ, :]
                                          + rsb_recv_ref[s - 1, :, :])
                rs = pltpu.make_async_remote_copy(
                    src_ref=rsb_send_ref, dst_ref=rsb_recv_ref.at[s],
                    send_sem=rsb_send_sems.at[s],
                    recv_sem=rsb_recv_sems.at[s],
                    device_id=(left,), device_id_type=pl.DeviceIdType.MESH)
                rs.start()
                rsb_rdmas[s] = rs
            else:
                HB = HR // 2
                for rb in range(2):
                    lo = rb * HB
                    attn_block(xgb_ref[h - 1, lo:lo + HB, :], acc_ref,
                               HR + lo, HB)
                    if rb == 0:
                        prev = pltpu.make_async_remote_copy(
                            src_ref=rsb_send_ref,
                            dst_ref=rsb_recv_ref.at[s - 1],
                            send_sem=rsb_send_sems.at[s - 1],
                            recv_sem=rsb_recv_sems.at[s - 1],
                            device_id=(right,),
                            device_id_type=pl.DeviceIdType.MESH)
                        prev.wait_recv()
                    fbf_send_ref[lo:lo + HB, :] = (
                        acc_ref[HR + lo:HR + lo + HB, :]
                        + rsb_recv_ref[s - 1, lo:lo + HB, :]
                    ).astype(jnp.bfloat16)
                    half = pltpu.make_async_remote_copy(
                        src_ref=fbf_send_ref.at[pl.ds(lo, HB)],
                        dst_ref=fbf_recv_ref.at[pl.ds(lo, HB)],
                        send_sem=rsb_send_sems.at[s + rb],
                        recv_sem=rsb_recv_sems.at[s + rb],
                        device_id=(left,),
                        device_id_type=pl.DeviceIdType.MESH)
                    half.start()
                    rsb_rdmas.append(half)

        fin = pltpu.make_async_remote_copy(
            src_ref=rsa_send_ref, dst_ref=rsa_recv_ref.at[N_DEV - 2],
            send_sem=rsa_send_sems.at[N_DEV - 2],
            recv_sem=rsa_recv_sems.at[N_DEV - 2],
            device_id=(left,), device_id_type=pl.DeviceIdType.MESH)
        fin.wait_recv()
        out_ref[0:HR, :] = (rsa_recv_ref[N_DEV - 2, :, :]
                            + pown_ref[0:HR, :])
        HB = HR // 2
        for rb in range(2):
            lo = rb * HB
            fin = pltpu.make_async_remote_copy(
                src_ref=fbf_send_ref.at[pl.ds(lo, HB)],
                dst_ref=fbf_recv_ref.at[pl.ds(lo, HB)],
                send_sem=rsb_send_sems.at[N_DEV - 2 + rb],
                recv_sem=rsb_recv_sems.at[N_DEV - 2 + rb],
                device_id=(right,), device_id_type=pl.DeviceIdType.MESH)
            fin.wait_recv()
            out_ref[HR + lo:HR + lo + HB, :] = (
                fbf_recv_ref[lo:lo + HB, :].astype(jnp.float32)
                + pown_ref[HR + lo:HR + lo + HB, :])

        rsa_rdmas[N_DEV - 2].wait_send()
        rsb_rdmas[N_DEV - 3].wait_send()
        rsb_rdmas[-2].wait_send()
        rsb_rdmas[-1].wait_send()
        for snd in sends:
            snd.wait_send()

        @functools.partial(pl.run_scoped,
                           second_barrier=pltpu.SemaphoreType.REGULAR)
        def _(second_barrier):
            for nbr in (left, right):
                pl.semaphore_signal(second_barrier, inc=1, device_id=(nbr,),
                                    device_id_type=pl.DeviceIdType.MESH)
            pl.semaphore_wait(second_barrier, 2)

    out = pl.pallas_call(
        body,
        out_shape=jax.ShapeDtypeStruct((SQ, D), jnp.float32),
        in_specs=[
            pl.BlockSpec(memory_space=pltpu.VMEM),
            pl.BlockSpec(memory_space=pltpu.VMEM),
            pl.BlockSpec(memory_space=pltpu.VMEM),
            pl.BlockSpec(memory_space=pl.ANY),
            pl.BlockSpec(memory_space=pl.ANY),
        ],
        out_specs=pl.BlockSpec(memory_space=pltpu.VMEM),
        scratch_shapes=[
            pltpu.VMEM((N_DEV - 1, HR, D), jnp.float32),
            pltpu.VMEM((N_DEV - 1, HR, D), jnp.float32),
            pltpu.VMEM((SQ, D), jnp.float32),
            pltpu.VMEM((SQ, D), jnp.float32),
            pltpu.VMEM((SQ, D), jnp.float32),
            pltpu.VMEM((HR, D), jnp.float32),
            pltpu.VMEM((HR, D), jnp.float32),
            pltpu.VMEM((N_DEV - 1, HR, D), jnp.float32),
            pltpu.VMEM((N_DEV - 1, HR, D), jnp.float32),
            pltpu.VMEM((HR, D), jnp.bfloat16),
            pltpu.VMEM((HR, D), jnp.bfloat16),
            pltpu.VMEM((4, SKV, DH), jnp.float32),
            pltpu.VMEM((4, SKV, DH), jnp.float32),
            pltpu.SemaphoreType.DMA((N_DEV - 1,)),
            pltpu.SemaphoreType.DMA((N_DEV - 1,)),
            pltpu.SemaphoreType.DMA((N_DEV - 1,)),
            pltpu.SemaphoreType.DMA((N_DEV - 1,)),
            pltpu.SemaphoreType.DMA((N_DEV - 1,)),
            pltpu.SemaphoreType.DMA((N_DEV - 1,)),
            pltpu.SemaphoreType.DMA((N_DEV,)),
            pltpu.SemaphoreType.DMA((N_DEV,)),
            pltpu.SemaphoreType.DMA((4, 2)),
        ],
        compiler_params=pltpu.CompilerParams(collective_id=0),
    )(x2, Wq, Wo, K_ext, V_ext)
    return out.reshape(1, SQ, D)
